# baseline (speedup 1.0000x reference)
import functools

import numpy as np
import jax
import jax.numpy as jnp
from jax import lax
from jax.experimental import pallas as pl
from jax.experimental.pallas import tpu as pltpu
from jax.experimental.pallas import tpu_sc as plsc

_N_SEG = 64


def kernel(inp):
    b, n_t, d = inp.shape
    rows = b * _N_SEG

    info = plsc.get_sparse_core_info()
    num_workers = info.num_cores * info.num_subcores
    rpw = rows // num_workers
    hpw = rpw // 2
    wpb = _N_SEG // rpw

    table = inp.reshape(b * n_t, d)
    mesh = plsc.VectorSubcoreMesh(core_axis_name="c", subcore_axis_name="s")

    @functools.partial(
        pl.kernel,
        mesh=mesh,
        out_type=jax.ShapeDtypeStruct((b, _N_SEG, d), jnp.float32),
        scratch_types=[
            pltpu.VMEM((hpw,), jnp.int32),
            pltpu.VMEM((hpw,), jnp.int32),
            pltpu.VMEM((hpw, d), jnp.float32),
            pltpu.VMEM((hpw, d), jnp.float32),
            pltpu.SemaphoreType.DMA,
            pltpu.SemaphoreType.DMA,
            pltpu.SemaphoreType.DMA,
            pltpu.SemaphoreType.DMA,
        ],
    )
    def gather_rows(
        table_hbm, idx_hbm, out_hbm,
        idx_v0, idx_v1, rows_v0, rows_v1, s0, s1, s2, s3,
    ):
        wid = lax.axis_index("s") * info.num_cores + lax.axis_index("c")
        out_b = wid // wpb
        out_k = (wid % wpb) * rpw
        i0 = pltpu.async_copy(idx_hbm.at[wid, 0], idx_v0, s0)
        i1 = pltpu.async_copy(idx_hbm.at[wid, 1], idx_v1, s1)
        i0.wait()
        g0 = pltpu.async_copy(table_hbm.at[idx_v0], rows_v0, s2)
        i1.wait()
        g1 = pltpu.async_copy(table_hbm.at[idx_v1], rows_v1, s3)
        g0.wait()
        o0 = pltpu.async_copy(rows_v0, out_hbm.at[out_b, pl.ds(out_k, hpw)], s0)
        g1.wait()
        o1 = pltpu.async_copy(rows_v1, out_hbm.at[out_b, pl.ds(out_k + hpw, hpw)], s1)
        o0.wait()
        o1.wait()

    t_vec = np.linspace(1, n_t, _N_SEG + 1)
    starts = [int(round(x)) - 1 for x in t_vec[:-1]]
    flat_idx = np.asarray(
        [bi * n_t + t for bi in range(b) for t in starts], dtype=np.int32
    ).reshape(num_workers, 2, hpw)
    return gather_rows(table, jnp.asarray(flat_idx))


# speedup vs baseline: 1.0074x; 1.0074x over previous
import functools

import numpy as np
import jax
import jax.numpy as jnp
from jax import lax
from jax.experimental import pallas as pl
from jax.experimental.pallas import tpu as pltpu
from jax.experimental.pallas import tpu_sc as plsc

_N_SEG = 64


def kernel(inp):
    b, n_t, d = inp.shape
    rows = b * _N_SEG

    info = plsc.get_sparse_core_info()
    num_workers = 1 * info.num_subcores
    rpw = rows // num_workers
    hpw = rpw // 2
    wpb = _N_SEG // rpw

    table = inp.reshape(b * n_t, d)
    mesh = plsc.VectorSubcoreMesh(
        core_axis_name="c", subcore_axis_name="s", num_cores=1
    )

    @functools.partial(
        pl.kernel,
        mesh=mesh,
        out_type=jax.ShapeDtypeStruct((b, _N_SEG, d), jnp.float32),
        scratch_types=[
            pltpu.VMEM((hpw,), jnp.int32),
            pltpu.VMEM((hpw,), jnp.int32),
            pltpu.VMEM((hpw, d), jnp.float32),
            pltpu.VMEM((hpw, d), jnp.float32),
            pltpu.SemaphoreType.DMA,
            pltpu.SemaphoreType.DMA,
            pltpu.SemaphoreType.DMA,
            pltpu.SemaphoreType.DMA,
        ],
    )
    def gather_rows(
        table_hbm, idx_hbm, out_hbm,
        idx_v0, idx_v1, rows_v0, rows_v1, s0, s1, s2, s3,
    ):
        wid = lax.axis_index("s")
        out_b = wid // wpb
        out_k = (wid % wpb) * rpw
        i0 = pltpu.async_copy(idx_hbm.at[wid, 0], idx_v0, s0)
        i1 = pltpu.async_copy(idx_hbm.at[wid, 1], idx_v1, s1)
        i0.wait()
        g0 = pltpu.async_copy(table_hbm.at[idx_v0], rows_v0, s2)
        i1.wait()
        g1 = pltpu.async_copy(table_hbm.at[idx_v1], rows_v1, s3)
        g0.wait()
        o0 = pltpu.async_copy(rows_v0, out_hbm.at[out_b, pl.ds(out_k, hpw)], s0)
        g1.wait()
        o1 = pltpu.async_copy(rows_v1, out_hbm.at[out_b, pl.ds(out_k + hpw, hpw)], s1)
        o0.wait()
        o1.wait()

    t_vec = np.linspace(1, n_t, _N_SEG + 1)
    starts = [int(round(x)) - 1 for x in t_vec[:-1]]
    flat_idx = np.asarray(
        [bi * n_t + t for bi in range(b) for t in starts], dtype=np.int32
    ).reshape(num_workers, 2, hpw)
    return gather_rows(table, jnp.asarray(flat_idx))
